# Initial kernel scaffold; baseline (speedup 1.0000x reference)
import jax
import jax.numpy as jnp
from jax.experimental import pallas as pl


def kernel(x, edge_index, W1, b1, W2, b2):
    raise NotImplementedError("write your pallas kernel here")


# speedup vs baseline: 28.2502x; 28.2502x over previous
import functools

import jax
import jax.numpy as jnp
from jax import lax
from jax.experimental import pallas as pl
from jax.experimental.pallas import tpu as pltpu
from jax.experimental.pallas import tpu_sc as plsc

NC = 2
NS = 16
NW = NC * NS
CHUNK = 128


def _sc_degree(dstw, zeros1, np_rows, k):
    stripe = np_rows // NS
    mesh = plsc.VectorSubcoreMesh(core_axis_name="c", subcore_axis_name="s")

    @functools.partial(
        pl.kernel,
        mesh=mesh,
        compiler_params=pltpu.CompilerParams(use_tc_tiling_on_sc=False),
        out_type=jax.ShapeDtypeStruct((NC, np_rows), jnp.float32),
        scratch_types=[
            pltpu.VMEM((k, CHUNK), jnp.int32),
            pltpu.VMEM((CHUNK,), jnp.float32),
            pltpu.VMEM_SHARED((np_rows,), jnp.float32),
            pltpu.SemaphoreType.DMA,
        ],
    )
    def kfn(zeros_hbm, dstw_hbm, out_hbm, dst_v, ones_v, acc_sh, sem):
        c = lax.axis_index("c")
        s = lax.axis_index("s")
        w = c * NS + s
        pltpu.sync_copy(zeros_hbm.at[pl.ds(s * stripe, stripe)],
                        acc_sh.at[pl.ds(s * stripe, stripe)])
        for i in range(CHUNK // 16):
            ones_v[pl.ds(i * 16, 16)] = jnp.ones((16,), jnp.float32)
        pltpu.sync_copy(dstw_hbm.at[w], dst_v)
        plsc.subcore_barrier()

        def body(j, carry):
            pltpu.sync_copy(ones_v, acc_sh.at[dst_v.at[j]], add=True)
            return carry

        lax.fori_loop(0, k, body, 0)
        plsc.subcore_barrier()
        pltpu.sync_copy(acc_sh.at[pl.ds(s * stripe, stripe)],
                        out_hbm.at[c, pl.ds(s * stripe, stripe)])

    return kfn(zeros1, dstw)


def _sc_agg(y, srcw, dstw, zerosd, np_rows, k, d):
    stripe = np_rows // NS
    mesh = plsc.VectorSubcoreMesh(core_axis_name="c", subcore_axis_name="s")

    @functools.partial(
        pl.kernel,
        mesh=mesh,
        compiler_params=pltpu.CompilerParams(use_tc_tiling_on_sc=False),
        out_type=jax.ShapeDtypeStruct((NC, np_rows, d), jnp.float32),
        scratch_types=[
            pltpu.VMEM((k, CHUNK), jnp.int32),
            pltpu.VMEM((k, CHUNK), jnp.int32),
            pltpu.VMEM((CHUNK, d), jnp.float32),
            pltpu.VMEM_SHARED((np_rows, d), jnp.float32),
            pltpu.SemaphoreType.DMA,
        ],
    )
    def kfn(y_hbm, srcw_hbm, dstw_hbm, zeros_hbm, out_hbm,
            src_v, dst_v, rows_v, acc_sh, sem):
        c = lax.axis_index("c")
        s = lax.axis_index("s")
        w = c * NS + s
        pltpu.sync_copy(zeros_hbm.at[pl.ds(s * stripe, stripe)],
                        acc_sh.at[pl.ds(s * stripe, stripe)])
        pltpu.sync_copy(srcw_hbm.at[w], src_v)
        pltpu.sync_copy(dstw_hbm.at[w], dst_v)
        plsc.subcore_barrier()

        def body(j, carry):
            pltpu.async_copy(y_hbm.at[src_v.at[j]], rows_v, sem).wait()
            pltpu.sync_copy(rows_v, acc_sh.at[dst_v.at[j]], add=True)
            return carry

        lax.fori_loop(0, k, body, 0)
        plsc.subcore_barrier()
        pltpu.sync_copy(acc_sh.at[pl.ds(s * stripe, stripe)],
                        out_hbm.at[c, pl.ds(s * stripe, stripe)])

    return kfn(y, srcw, dstw, zerosd)


def _tc_prep(x, degp_t, W1, bn):
    n, din = x.shape
    hid = W1.shape[1]

    def body(x_ref, degp_ref, w1_ref, y1_ref, dinv_ref):
        deg = degp_ref[:, 0:1] + degp_ref[:, 1:2] + 1.0
        dinv = lax.rsqrt(deg)
        xw = jnp.dot(x_ref[...], w1_ref[...],
                     preferred_element_type=jnp.float32)
        y1_ref[...] = xw * dinv
        dinv_ref[...] = dinv

    return pl.pallas_call(
        body,
        grid=(n // bn,),
        in_specs=[
            pl.BlockSpec((bn, din), lambda i: (i, 0)),
            pl.BlockSpec((bn, 2), lambda i: (i, 0)),
            pl.BlockSpec((din, hid), lambda i: (0, 0)),
        ],
        out_specs=[
            pl.BlockSpec((bn, hid), lambda i: (i, 0)),
            pl.BlockSpec((bn, 1), lambda i: (i, 0)),
        ],
        out_shape=[
            jax.ShapeDtypeStruct((n, hid), jnp.float32),
            jax.ShapeDtypeStruct((n, 1), jnp.float32),
        ],
    )(x, degp_t, W1)


def _tc_layer1(aggp, y1, dinv, b1, W2, bn):
    n, hid = y1.shape
    c = W2.shape[1]

    def body(aggp_ref, y1_ref, dinv_ref, b1_ref, w2_ref, y2_ref):
        ssum = aggp_ref[0] + aggp_ref[1] + y1_ref[...]
        h = jnp.maximum(ssum * dinv_ref[...] + b1_ref[...], 0.0)
        y2_ref[...] = jnp.dot(h, w2_ref[...],
                              preferred_element_type=jnp.float32) * dinv_ref[...]

    return pl.pallas_call(
        body,
        grid=(n // bn,),
        in_specs=[
            pl.BlockSpec((2, bn, hid), lambda i: (0, i, 0)),
            pl.BlockSpec((bn, hid), lambda i: (i, 0)),
            pl.BlockSpec((bn, 1), lambda i: (i, 0)),
            pl.BlockSpec((1, hid), lambda i: (0, 0)),
            pl.BlockSpec((hid, c), lambda i: (0, 0)),
        ],
        out_specs=pl.BlockSpec((bn, c), lambda i: (i, 0)),
        out_shape=jax.ShapeDtypeStruct((n, c), jnp.float32),
    )(aggp, y1, dinv, b1, W2)


def _tc_out(aggp, y2, dinv, b2, bn):
    n, c = y2.shape

    def body(aggp_ref, y2_ref, dinv_ref, b2_ref, out_ref):
        o = (aggp_ref[0] + aggp_ref[1] + y2_ref[...]) * dinv_ref[...] \
            + b2_ref[...]
        m = jnp.max(o, axis=1, keepdims=True)
        e = jnp.exp(o - m)
        lse = jnp.log(jnp.sum(e, axis=1, keepdims=True))
        out_ref[...] = o - m - lse

    return pl.pallas_call(
        body,
        grid=(n // bn,),
        in_specs=[
            pl.BlockSpec((2, bn, c), lambda i: (0, i, 0)),
            pl.BlockSpec((bn, c), lambda i: (i, 0)),
            pl.BlockSpec((bn, 1), lambda i: (i, 0)),
            pl.BlockSpec((1, c), lambda i: (0, 0)),
        ],
        out_specs=pl.BlockSpec((bn, c), lambda i: (i, 0)),
        out_shape=jax.ShapeDtypeStruct((n, c), jnp.float32),
    )(aggp, y2, dinv, b2)


def kernel(x, edge_index, W1, b1, W2, b2):
    n, din = x.shape
    hid = W1.shape[1]
    nc = W2.shape[1]
    e = edge_index.shape[1]

    src = edge_index[0].astype(jnp.int32)
    dst = edge_index[1].astype(jnp.int32)

    k = -(-e // (NW * CHUNK))
    tot = NW * CHUNK * k
    pad = tot - e
    np_rows = -(-(n + 1) // 256) * 256

    srcw = jnp.concatenate(
        [src, jnp.zeros((pad,), jnp.int32)]).reshape(NW, k, CHUNK)
    dstw = jnp.concatenate(
        [dst, jnp.full((pad,), n, jnp.int32)]).reshape(NW, k, CHUNK)
    zeros1 = jnp.zeros((np_rows,), jnp.float32)
    zerosh = jnp.zeros((np_rows, hid), jnp.float32)
    zerosc = jnp.zeros((np_rows, nc), jnp.float32)

    bn = 1000 if n % 1000 == 0 else 8

    degp = _sc_degree(dstw, zeros1, np_rows, k)
    degp_t = degp[:, :n].T
    y1, dinv = _tc_prep(x, degp_t, W1, bn)
    agg1 = _sc_agg(y1, srcw, dstw, zerosh, np_rows, k, hid)
    y2 = _tc_layer1(agg1[:, :n], y1, dinv,
                    b1.reshape(1, hid), W2, bn)
    agg2 = _sc_agg(y2, srcw, dstw, zerosc, np_rows, k, nc)
    return _tc_out(agg2[:, :n], y2, dinv, b2.reshape(1, nc), bn)


# speedup vs baseline: 30.6208x; 1.0839x over previous
import functools

import jax
import jax.numpy as jnp
from jax import lax
from jax.experimental import pallas as pl
from jax.experimental.pallas import tpu as pltpu
from jax.experimental.pallas import tpu_sc as plsc

NC = 2
NS = 16
NW = NC * NS
CHUNK = 128
D = 4
A = 2


def _sc_degree(dstw, zeros1, np_rows, k):
    stripe = np_rows // NS
    mesh = plsc.VectorSubcoreMesh(core_axis_name="c", subcore_axis_name="s")

    @functools.partial(
        pl.kernel,
        mesh=mesh,
        compiler_params=pltpu.CompilerParams(use_tc_tiling_on_sc=False),
        out_type=jax.ShapeDtypeStruct((NC, np_rows), jnp.float32),
        scratch_types=[
            pltpu.VMEM((k, CHUNK), jnp.int32),
            pltpu.VMEM((CHUNK,), jnp.float32),
            pltpu.VMEM_SHARED((np_rows,), jnp.float32),
        ] + [pltpu.SemaphoreType.DMA] * D,
    )
    def kfn(zeros_hbm, dstw_hbm, out_hbm, dst_v, ones_v, acc_sh, *sems):
        c = lax.axis_index("c")
        s = lax.axis_index("s")
        w = c * NS + s
        pltpu.sync_copy(zeros_hbm.at[pl.ds(s * stripe, stripe)],
                        acc_sh.at[pl.ds(s * stripe, stripe)])
        for i in range(CHUNK // 16):
            ones_v[pl.ds(i * 16, 16)] = jnp.ones((16,), jnp.float32)
        pltpu.sync_copy(dstw_hbm.at[w], dst_v)
        plsc.subcore_barrier()

        def start(i, b):
            pltpu.async_copy(ones_v, acc_sh.at[dst_v.at[i]], sems[b],
                             add=True)

        def drain(b):
            pltpu.make_async_copy(ones_v, acc_sh.at[dst_v.at[0]],
                                  sems[b]).wait()

        for i in range(D):
            start(i, i)

        def group(jo, carry):
            for b in range(D):
                i = jo * D + b
                drain(b)
                start(i, b)
            return carry

        lax.fori_loop(1, k // D, group, 0)
        for b in range(D):
            drain(b)
        plsc.subcore_barrier()
        pltpu.sync_copy(acc_sh.at[pl.ds(s * stripe, stripe)],
                        out_hbm.at[c, pl.ds(s * stripe, stripe)])

    return kfn(zeros1, dstw)


def _sc_agg(y, srcw, dstw, zerosd, np_rows, k, d):
    stripe = np_rows // NS
    mesh = plsc.VectorSubcoreMesh(core_axis_name="c", subcore_axis_name="s")

    @functools.partial(
        pl.kernel,
        mesh=mesh,
        compiler_params=pltpu.CompilerParams(use_tc_tiling_on_sc=False),
        out_type=jax.ShapeDtypeStruct((NC, np_rows, d), jnp.float32),
        scratch_types=[
            pltpu.VMEM((k, CHUNK), jnp.int32),
            pltpu.VMEM((k, CHUNK), jnp.int32),
        ] + [pltpu.VMEM((CHUNK, d), jnp.float32)] * D
          + [pltpu.VMEM_SHARED((np_rows, d), jnp.float32)]
          + [pltpu.SemaphoreType.DMA] * (2 * D),
    )
    def kfn(y_hbm, srcw_hbm, dstw_hbm, zeros_hbm, out_hbm,
            src_v, dst_v, *bufs_and_sems):
        rows = bufs_and_sems[:D]
        acc_sh = bufs_and_sems[D]
        gsem = bufs_and_sems[D + 1:D + 1 + D]
        ssem = bufs_and_sems[D + 1 + D:D + 1 + 2 * D]
        c = lax.axis_index("c")
        s = lax.axis_index("s")
        w = c * NS + s
        pltpu.sync_copy(zeros_hbm.at[pl.ds(s * stripe, stripe)],
                        acc_sh.at[pl.ds(s * stripe, stripe)])
        pltpu.sync_copy(srcw_hbm.at[w], src_v)
        pltpu.sync_copy(dstw_hbm.at[w], dst_v)
        plsc.subcore_barrier()

        def start_gather(i, b):
            pltpu.async_copy(y_hbm.at[src_v.at[i]], rows[b], gsem[b])

        def wait_gather(i, b):
            pltpu.make_async_copy(y_hbm.at[src_v.at[i]], rows[b],
                                  gsem[b]).wait()

        def start_scatter(i, b):
            pltpu.async_copy(rows[b], acc_sh.at[dst_v.at[i]], ssem[b],
                             add=True)

        def drain_scatter(b):
            pltpu.make_async_copy(rows[b], acc_sh.at[dst_v.at[0]],
                                  ssem[b]).wait()

        for i in range(D):
            start_gather(i, i)
            if i >= A:
                wait_gather(i - A, i - A)
                start_scatter(i - A, i - A)

        def group(jo, carry):
            for b in range(D):
                i = jo * D + b
                drain_scatter(b)
                start_gather(i, b)
                bA = (b - A) % D
                wait_gather(i - A, bA)
                start_scatter(i - A, bA)
            return carry

        lax.fori_loop(1, k // D, group, 0)
        for t in range(A):
            i = k - A + t
            b = i % D
            wait_gather(i, b)
            start_scatter(i, b)
        for b in range(D):
            drain_scatter(b)
        plsc.subcore_barrier()
        pltpu.sync_copy(acc_sh.at[pl.ds(s * stripe, stripe)],
                        out_hbm.at[c, pl.ds(s * stripe, stripe)])

    return kfn(y, srcw, dstw, zerosd)


def _tc_prep(x, degp_t, W1, bn):
    n, din = x.shape
    hid = W1.shape[1]

    def body(x_ref, degp_ref, w1_ref, y1_ref, dinv_ref):
        deg = degp_ref[:, 0:1] + degp_ref[:, 1:2] + 1.0
        dinv = lax.rsqrt(deg)
        xw = jnp.dot(x_ref[...], w1_ref[...],
                     preferred_element_type=jnp.float32)
        y1_ref[...] = xw * dinv
        dinv_ref[...] = dinv

    return pl.pallas_call(
        body,
        grid=(n // bn,),
        in_specs=[
            pl.BlockSpec((bn, din), lambda i: (i, 0)),
            pl.BlockSpec((bn, 2), lambda i: (i, 0)),
            pl.BlockSpec((din, hid), lambda i: (0, 0)),
        ],
        out_specs=[
            pl.BlockSpec((bn, hid), lambda i: (i, 0)),
            pl.BlockSpec((bn, 1), lambda i: (i, 0)),
        ],
        out_shape=[
            jax.ShapeDtypeStruct((n, hid), jnp.float32),
            jax.ShapeDtypeStruct((n, 1), jnp.float32),
        ],
    )(x, degp_t, W1)


def _tc_layer1(aggp, y1, dinv, b1, W2, bn):
    n, hid = y1.shape
    c = W2.shape[1]

    def body(aggp_ref, y1_ref, dinv_ref, b1_ref, w2_ref, y2_ref):
        ssum = aggp_ref[0] + aggp_ref[1] + y1_ref[...]
        h = jnp.maximum(ssum * dinv_ref[...] + b1_ref[...], 0.0)
        y2_ref[...] = jnp.dot(h, w2_ref[...],
                              preferred_element_type=jnp.float32) * dinv_ref[...]

    return pl.pallas_call(
        body,
        grid=(n // bn,),
        in_specs=[
            pl.BlockSpec((2, bn, hid), lambda i: (0, i, 0)),
            pl.BlockSpec((bn, hid), lambda i: (i, 0)),
            pl.BlockSpec((bn, 1), lambda i: (i, 0)),
            pl.BlockSpec((1, hid), lambda i: (0, 0)),
            pl.BlockSpec((hid, c), lambda i: (0, 0)),
        ],
        out_specs=pl.BlockSpec((bn, c), lambda i: (i, 0)),
        out_shape=jax.ShapeDtypeStruct((n, c), jnp.float32),
    )(aggp, y1, dinv, b1, W2)


def _tc_out(aggp, y2, dinv, b2, bn):
    n, c = y2.shape

    def body(aggp_ref, y2_ref, dinv_ref, b2_ref, out_ref):
        o = (aggp_ref[0] + aggp_ref[1] + y2_ref[...]) * dinv_ref[...] \
            + b2_ref[...]
        m = jnp.max(o, axis=1, keepdims=True)
        e = jnp.exp(o - m)
        lse = jnp.log(jnp.sum(e, axis=1, keepdims=True))
        out_ref[...] = o - m - lse

    return pl.pallas_call(
        body,
        grid=(n // bn,),
        in_specs=[
            pl.BlockSpec((2, bn, c), lambda i: (0, i, 0)),
            pl.BlockSpec((bn, c), lambda i: (i, 0)),
            pl.BlockSpec((bn, 1), lambda i: (i, 0)),
            pl.BlockSpec((1, c), lambda i: (0, 0)),
        ],
        out_specs=pl.BlockSpec((bn, c), lambda i: (i, 0)),
        out_shape=jax.ShapeDtypeStruct((n, c), jnp.float32),
    )(aggp, y2, dinv, b2)


def kernel(x, edge_index, W1, b1, W2, b2):
    n, din = x.shape
    hid = W1.shape[1]
    nc = W2.shape[1]
    e = edge_index.shape[1]

    src = edge_index[0].astype(jnp.int32)
    dst = edge_index[1].astype(jnp.int32)

    k = -(-e // (NW * CHUNK))
    k = max(-(-k // D) * D, 2 * D)
    tot = NW * CHUNK * k
    pad = tot - e
    np_rows = -(-(n + 1) // 256) * 256

    srcw = jnp.concatenate(
        [src, jnp.zeros((pad,), jnp.int32)]).reshape(NW, k, CHUNK)
    dstw = jnp.concatenate(
        [dst, jnp.full((pad,), n, jnp.int32)]).reshape(NW, k, CHUNK)
    zeros1 = jnp.zeros((np_rows,), jnp.float32)
    zerosh = jnp.zeros((np_rows, hid), jnp.float32)
    zerosc = jnp.zeros((np_rows, nc), jnp.float32)

    bn = 1000 if n % 1000 == 0 else 8

    degp = _sc_degree(dstw, zeros1, np_rows, k)
    degp_t = degp[:, :n].T
    y1, dinv = _tc_prep(x, degp_t, W1, bn)
    agg1 = _sc_agg(y1, srcw, dstw, zerosh, np_rows, k, hid)
    y2 = _tc_layer1(agg1[:, :n], y1, dinv,
                    b1.reshape(1, hid), W2, bn)
    agg2 = _sc_agg(y2, srcw, dstw, zerosc, np_rows, k, nc)
    return _tc_out(agg2[:, :n], y2, dinv, b2.reshape(1, nc), bn)


# speedup vs baseline: 48.3977x; 1.5805x over previous
import functools

import jax
import jax.numpy as jnp
from jax import lax
from jax.experimental import pallas as pl
from jax.experimental.pallas import tpu as pltpu
from jax.experimental.pallas import tpu_sc as plsc

NC = 2
NS = 16
NW = NC * NS
CHUNK = 128
D = 4
A = 2


def _sc_degree(dstw, zeros1, np_rows, k):
    stripe = np_rows // NS
    mesh = plsc.VectorSubcoreMesh(core_axis_name="c", subcore_axis_name="s")

    @functools.partial(
        pl.kernel,
        mesh=mesh,
        compiler_params=pltpu.CompilerParams(use_tc_tiling_on_sc=False),
        out_type=jax.ShapeDtypeStruct((NC, np_rows), jnp.float32),
        scratch_types=[
            pltpu.VMEM((k, CHUNK), jnp.int32),
            pltpu.VMEM((CHUNK,), jnp.float32),
            pltpu.VMEM_SHARED((np_rows,), jnp.float32),
        ] + [pltpu.SemaphoreType.DMA] * D,
    )
    def kfn(zeros_hbm, dstw_hbm, out_hbm, dst_v, ones_v, acc_sh, *sems):
        c = lax.axis_index("c")
        s = lax.axis_index("s")
        w = c * NS + s
        pltpu.sync_copy(zeros_hbm.at[pl.ds(s * stripe, stripe)],
                        acc_sh.at[pl.ds(s * stripe, stripe)])
        for i in range(CHUNK // 16):
            ones_v[pl.ds(i * 16, 16)] = jnp.ones((16,), jnp.float32)
        pltpu.sync_copy(dstw_hbm.at[w], dst_v)
        plsc.subcore_barrier()

        def start(i, b):
            pltpu.async_copy(ones_v, acc_sh.at[dst_v.at[i]], sems[b],
                             add=True)

        def drain(b):
            pltpu.make_async_copy(ones_v, acc_sh.at[dst_v.at[0]],
                                  sems[b]).wait()

        for i in range(D):
            start(i, i)

        def group(jo, carry):
            for b in range(D):
                i = jo * D + b
                drain(b)
                start(i, b)
            return carry

        lax.fori_loop(1, k // D, group, 0)
        for b in range(D):
            drain(b)
        plsc.subcore_barrier()
        pltpu.sync_copy(acc_sh.at[pl.ds(s * stripe, stripe)],
                        out_hbm.at[c, pl.ds(s * stripe, stripe)])

    return kfn(zeros1, dstw)


def _sc_agg(y, srcw, dstw, zerosd, np_rows, k, d):
    stripe = np_rows // NS
    mesh = plsc.VectorSubcoreMesh(core_axis_name="c", subcore_axis_name="s")

    @functools.partial(
        pl.kernel,
        mesh=mesh,
        compiler_params=pltpu.CompilerParams(use_tc_tiling_on_sc=False),
        out_type=jax.ShapeDtypeStruct((NC, np_rows, d), jnp.float32),
        scratch_types=[
            pltpu.VMEM((k, CHUNK), jnp.int32),
            pltpu.VMEM((k, CHUNK), jnp.int32),
        ] + [pltpu.VMEM((CHUNK, d), jnp.float32)] * D
          + [pltpu.VMEM_SHARED((np_rows, d), jnp.float32)] * 2
          + [pltpu.SemaphoreType.DMA] * (2 * D),
    )
    def kfn(y_hbm, srcw_hbm, dstw_hbm, zeros_hbm, out_hbm,
            src_v, dst_v, *bufs_and_sems):
        rows = bufs_and_sems[:D]
        acc_sh = bufs_and_sems[D]
        y_sh = bufs_and_sems[D + 1]
        gsem = bufs_and_sems[D + 2:D + 2 + D]
        ssem = bufs_and_sems[D + 2 + D:D + 2 + 2 * D]
        n = y_hbm.shape[0]
        ystripe = n // NS
        c = lax.axis_index("c")
        s = lax.axis_index("s")
        w = c * NS + s
        pltpu.sync_copy(zeros_hbm.at[pl.ds(s * stripe, stripe)],
                        acc_sh.at[pl.ds(s * stripe, stripe)])
        pltpu.sync_copy(y_hbm.at[pl.ds(s * ystripe, ystripe)],
                        y_sh.at[pl.ds(s * ystripe, ystripe)])
        pltpu.sync_copy(srcw_hbm.at[w], src_v)
        pltpu.sync_copy(dstw_hbm.at[w], dst_v)
        plsc.subcore_barrier()

        def start_gather(i, b):
            pltpu.async_copy(y_sh.at[src_v.at[i]], rows[b], gsem[b])

        def wait_gather(i, b):
            pltpu.make_async_copy(y_sh.at[src_v.at[i]], rows[b],
                                  gsem[b]).wait()

        def start_scatter(i, b):
            pltpu.async_copy(rows[b], acc_sh.at[dst_v.at[i]], ssem[b],
                             add=True)

        def drain_scatter(b):
            pltpu.make_async_copy(rows[b], acc_sh.at[dst_v.at[0]],
                                  ssem[b]).wait()

        for i in range(D):
            start_gather(i, i)
            if i >= A:
                wait_gather(i - A, i - A)
                start_scatter(i - A, i - A)

        def group(jo, carry):
            for b in range(D):
                i = jo * D + b
                drain_scatter(b)
                start_gather(i, b)
                bA = (b - A) % D
                wait_gather(i - A, bA)
                start_scatter(i - A, bA)
            return carry

        lax.fori_loop(1, k // D, group, 0)
        for t in range(A):
            i = k - A + t
            b = i % D
            wait_gather(i, b)
            start_scatter(i, b)
        for b in range(D):
            drain_scatter(b)
        plsc.subcore_barrier()
        pltpu.sync_copy(acc_sh.at[pl.ds(s * stripe, stripe)],
                        out_hbm.at[c, pl.ds(s * stripe, stripe)])

    return kfn(y, srcw, dstw, zerosd)


def _tc_prep(x, degp_t, W1, bn):
    n, din = x.shape
    hid = W1.shape[1]

    def body(x_ref, degp_ref, w1_ref, y1_ref, dinv_ref):
        deg = degp_ref[:, 0:1] + degp_ref[:, 1:2] + 1.0
        dinv = lax.rsqrt(deg)
        xw = jnp.dot(x_ref[...], w1_ref[...],
                     preferred_element_type=jnp.float32)
        y1_ref[...] = xw * dinv
        dinv_ref[...] = dinv

    return pl.pallas_call(
        body,
        grid=(n // bn,),
        in_specs=[
            pl.BlockSpec((bn, din), lambda i: (i, 0)),
            pl.BlockSpec((bn, 2), lambda i: (i, 0)),
            pl.BlockSpec((din, hid), lambda i: (0, 0)),
        ],
        out_specs=[
            pl.BlockSpec((bn, hid), lambda i: (i, 0)),
            pl.BlockSpec((bn, 1), lambda i: (i, 0)),
        ],
        out_shape=[
            jax.ShapeDtypeStruct((n, hid), jnp.float32),
            jax.ShapeDtypeStruct((n, 1), jnp.float32),
        ],
    )(x, degp_t, W1)


def _tc_layer1(aggp, y1, dinv, b1, W2, bn):
    n, hid = y1.shape
    c = W2.shape[1]

    def body(aggp_ref, y1_ref, dinv_ref, b1_ref, w2_ref, y2_ref):
        ssum = aggp_ref[0] + aggp_ref[1] + y1_ref[...]
        h = jnp.maximum(ssum * dinv_ref[...] + b1_ref[...], 0.0)
        y2_ref[...] = jnp.dot(h, w2_ref[...],
                              preferred_element_type=jnp.float32) * dinv_ref[...]

    return pl.pallas_call(
        body,
        grid=(n // bn,),
        in_specs=[
            pl.BlockSpec((2, bn, hid), lambda i: (0, i, 0)),
            pl.BlockSpec((bn, hid), lambda i: (i, 0)),
            pl.BlockSpec((bn, 1), lambda i: (i, 0)),
            pl.BlockSpec((1, hid), lambda i: (0, 0)),
            pl.BlockSpec((hid, c), lambda i: (0, 0)),
        ],
        out_specs=pl.BlockSpec((bn, c), lambda i: (i, 0)),
        out_shape=jax.ShapeDtypeStruct((n, c), jnp.float32),
    )(aggp, y1, dinv, b1, W2)


def _tc_out(aggp, y2, dinv, b2, bn):
    n, c = y2.shape

    def body(aggp_ref, y2_ref, dinv_ref, b2_ref, out_ref):
        o = (aggp_ref[0] + aggp_ref[1] + y2_ref[...]) * dinv_ref[...] \
            + b2_ref[...]
        m = jnp.max(o, axis=1, keepdims=True)
        e = jnp.exp(o - m)
        lse = jnp.log(jnp.sum(e, axis=1, keepdims=True))
        out_ref[...] = o - m - lse

    return pl.pallas_call(
        body,
        grid=(n // bn,),
        in_specs=[
            pl.BlockSpec((2, bn, c), lambda i: (0, i, 0)),
            pl.BlockSpec((bn, c), lambda i: (i, 0)),
            pl.BlockSpec((bn, 1), lambda i: (i, 0)),
            pl.BlockSpec((1, c), lambda i: (0, 0)),
        ],
        out_specs=pl.BlockSpec((bn, c), lambda i: (i, 0)),
        out_shape=jax.ShapeDtypeStruct((n, c), jnp.float32),
    )(aggp, y2, dinv, b2)


def kernel(x, edge_index, W1, b1, W2, b2):
    n, din = x.shape
    hid = W1.shape[1]
    nc = W2.shape[1]
    e = edge_index.shape[1]

    src = edge_index[0].astype(jnp.int32)
    dst = edge_index[1].astype(jnp.int32)

    k = -(-e // (NW * CHUNK))
    k = max(-(-k // D) * D, 2 * D)
    tot = NW * CHUNK * k
    pad = tot - e
    np_rows = -(-(n + 1) // 256) * 256

    srcw = jnp.concatenate(
        [src, jnp.zeros((pad,), jnp.int32)]).reshape(NW, k, CHUNK)
    dstw = jnp.concatenate(
        [dst, jnp.full((pad,), n, jnp.int32)]).reshape(NW, k, CHUNK)
    zeros1 = jnp.zeros((np_rows,), jnp.float32)
    zerosh = jnp.zeros((np_rows, hid), jnp.float32)
    zerosc = jnp.zeros((np_rows, nc), jnp.float32)

    bn = 1000 if n % 1000 == 0 else 8

    degp = _sc_degree(dstw, zeros1, np_rows, k)
    degp_t = degp[:, :n].T
    y1, dinv = _tc_prep(x, degp_t, W1, bn)
    agg1 = _sc_agg(y1, srcw, dstw, zerosh, np_rows, k, hid)
    y2 = _tc_layer1(agg1[:, :n], y1, dinv,
                    b1.reshape(1, hid), W2, bn)
    agg2 = _sc_agg(y2, srcw, dstw, zerosc, np_rows, k, nc)
    return _tc_out(agg2[:, :n], y2, dinv, b2.reshape(1, nc), bn)


# speedup vs baseline: 59.8118x; 1.2358x over previous
import functools

import jax
import jax.numpy as jnp
from jax import lax
from jax.experimental import pallas as pl
from jax.experimental.pallas import tpu as pltpu
from jax.experimental.pallas import tpu_sc as plsc

NC = 2
NS = 16
NW = NC * NS
CHUNK = 128


def _ring(base):
    for d in (8, 6, 4, 2):
        if base % d == 0 and base // d >= 2:
            return d
    return 1


def _sc_degree(e3, zeros1, np_rows, base, extra):
    stripe = np_rows // NS
    D = _ring(base)
    mesh = plsc.VectorSubcoreMesh(core_axis_name="c", subcore_axis_name="s")

    @functools.partial(
        pl.kernel,
        mesh=mesh,
        compiler_params=pltpu.CompilerParams(use_tc_tiling_on_sc=False),
        out_type=jax.ShapeDtypeStruct((NC, np_rows), jnp.float32),
        scratch_types=[
            pltpu.VMEM((base, CHUNK), jnp.int32),
            pltpu.VMEM((1, CHUNK), jnp.int32),
            pltpu.VMEM((CHUNK,), jnp.float32),
            pltpu.VMEM_SHARED((np_rows,), jnp.float32),
        ] + [pltpu.SemaphoreType.DMA] * max(D, 1),
    )
    def kfn(zeros_hbm, e3_hbm, out_hbm, dst_v, dstx_v, ones_v, acc_sh,
            *sems):
        c = lax.axis_index("c")
        s = lax.axis_index("s")
        w = c * NS + s
        pltpu.sync_copy(zeros_hbm.at[pl.ds(s * stripe, stripe)],
                        acc_sh.at[pl.ds(s * stripe, stripe)])
        for i in range(CHUNK // 16):
            ones_v[pl.ds(i * 16, 16)] = jnp.ones((16,), jnp.float32)
        pltpu.sync_copy(e3_hbm.at[1, pl.ds(w * base, base)], dst_v)
        plsc.subcore_barrier()

        def start(i, b):
            pltpu.async_copy(ones_v, acc_sh.at[dst_v.at[i]], sems[b],
                             add=True)

        def drain(b):
            pltpu.make_async_copy(ones_v, acc_sh.at[dst_v.at[0]],
                                  sems[b]).wait()

        for i in range(D):
            start(i, i)

        def group(jo, carry):
            for b in range(D):
                drain(b)
                start(jo * D + b, b)
            return carry

        lax.fori_loop(1, base // D, group, 0)
        for b in range(D):
            drain(b)
        @pl.when(w < extra)
        def _():
            pltpu.sync_copy(e3_hbm.at[1, pl.ds(base * NW + w, 1)], dstx_v)
            pltpu.sync_copy(ones_v, acc_sh.at[dstx_v.at[0]], add=True)

        plsc.subcore_barrier()
        pltpu.sync_copy(acc_sh.at[pl.ds(s * stripe, stripe)],
                        out_hbm.at[c, pl.ds(s * stripe, stripe)])

    return kfn(zeros1, e3)


def _sc_agg(y, e3, zerosd, np_rows, base, extra, d):
    stripe = np_rows // NS
    D = _ring(base)
    A = min(2, max(D - 1, 1))
    mesh = plsc.VectorSubcoreMesh(core_axis_name="c", subcore_axis_name="s")

    @functools.partial(
        pl.kernel,
        mesh=mesh,
        compiler_params=pltpu.CompilerParams(use_tc_tiling_on_sc=False),
        out_type=jax.ShapeDtypeStruct((NC, np_rows, d), jnp.float32),
        scratch_types=[
            pltpu.VMEM((base, CHUNK), jnp.int32),
            pltpu.VMEM((base, CHUNK), jnp.int32),
            pltpu.VMEM((1, CHUNK), jnp.int32),
            pltpu.VMEM((1, CHUNK), jnp.int32),
        ] + [pltpu.VMEM((CHUNK, d), jnp.float32)] * (max(D, 1) + 1)
          + [pltpu.VMEM_SHARED((np_rows, d), jnp.float32)] * 2
          + [pltpu.SemaphoreType.DMA] * (2 * max(D, 1) + 1),
    )
    def kfn(y_hbm, e3_hbm, zeros_hbm, out_hbm,
            src_v, dst_v, srcx_v, dstx_v, *bufs_and_sems):
        rows = bufs_and_sems[:D]
        rowsx = bufs_and_sems[D]
        acc_sh = bufs_and_sems[D + 1]
        y_sh = bufs_and_sems[D + 2]
        gsem = bufs_and_sems[D + 3:D + 3 + D]
        ssem = bufs_and_sems[D + 3 + D:D + 3 + 2 * D]
        xsem = bufs_and_sems[D + 3 + 2 * D]
        n = y_hbm.shape[0]
        ystripe = n // NS
        c = lax.axis_index("c")
        s = lax.axis_index("s")
        w = c * NS + s
        pltpu.sync_copy(zeros_hbm.at[pl.ds(s * stripe, stripe)],
                        acc_sh.at[pl.ds(s * stripe, stripe)])
        pltpu.sync_copy(y_hbm.at[pl.ds(s * ystripe, ystripe)],
                        y_sh.at[pl.ds(s * ystripe, ystripe)])
        pltpu.sync_copy(e3_hbm.at[0, pl.ds(w * base, base)], src_v)
        pltpu.sync_copy(e3_hbm.at[1, pl.ds(w * base, base)], dst_v)
        plsc.subcore_barrier()

        def start_gather(i, b):
            pltpu.async_copy(y_sh.at[src_v.at[i]], rows[b], gsem[b])

        def wait_gather(i, b):
            pltpu.make_async_copy(y_sh.at[src_v.at[i]], rows[b],
                                  gsem[b]).wait()

        def start_scatter(i, b):
            pltpu.async_copy(rows[b], acc_sh.at[dst_v.at[i]], ssem[b],
                             add=True)

        def drain_scatter(b):
            pltpu.make_async_copy(rows[b], acc_sh.at[dst_v.at[0]],
                                  ssem[b]).wait()

        for i in range(D):
            start_gather(i, i)
            if i >= A:
                wait_gather(i - A, i - A)
                start_scatter(i - A, i - A)

        def group(jo, carry):
            for b in range(D):
                i = jo * D + b
                drain_scatter(b)
                start_gather(i, b)
                bA = (b - A) % D
                wait_gather(i - A, bA)
                start_scatter(i - A, bA)
            return carry

        lax.fori_loop(1, base // D, group, 0)
        for t in range(A):
            i = base - A + t
            b = i % D
            wait_gather(i, b)
            start_scatter(i, b)
        for b in range(D):
            drain_scatter(b)
        @pl.when(w < extra)
        def _():
            pltpu.sync_copy(e3_hbm.at[0, pl.ds(base * NW + w, 1)], srcx_v)
            pltpu.sync_copy(e3_hbm.at[1, pl.ds(base * NW + w, 1)], dstx_v)
            pltpu.async_copy(y_sh.at[srcx_v.at[0]], rowsx, xsem).wait()
            pltpu.sync_copy(rowsx, acc_sh.at[dstx_v.at[0]], add=True)

        plsc.subcore_barrier()
        pltpu.sync_copy(acc_sh.at[pl.ds(s * stripe, stripe)],
                        out_hbm.at[c, pl.ds(s * stripe, stripe)])

    return kfn(y, e3, zerosd)


def _tc_prep(x, degp, W1, bn):
    n, din = x.shape
    hid = W1.shape[1]

    def body(x_ref, degp_ref, w1_ref, y1_ref, dinv_ref):
        deg = degp_ref[:, 0:1] + degp_ref[:, 1:2] + 1.0
        dinv = lax.rsqrt(deg)
        xw = jnp.dot(x_ref[...], w1_ref[...],
                     preferred_element_type=jnp.float32)
        y1_ref[...] = xw * dinv
        dinv_ref[...] = dinv

    return pl.pallas_call(
        body,
        grid=(n // bn,),
        in_specs=[
            pl.BlockSpec((bn, din), lambda i: (i, 0)),
            pl.BlockSpec((bn, 2), lambda i: (i, 0)),
            pl.BlockSpec((din, hid), lambda i: (0, 0)),
        ],
        out_specs=[
            pl.BlockSpec((bn, hid), lambda i: (i, 0)),
            pl.BlockSpec((bn, 1), lambda i: (i, 0)),
        ],
        out_shape=[
            jax.ShapeDtypeStruct((n, hid), jnp.float32),
            jax.ShapeDtypeStruct((n, 1), jnp.float32),
        ],
    )(x, degp, W1)


def _tc_layer1(aggp, y1, dinv, b1, W2, bn):
    n, hid = y1.shape
    c = W2.shape[1]

    def body(aggp_ref, y1_ref, dinv_ref, b1_ref, w2_ref, y2_ref):
        ssum = aggp_ref[0] + aggp_ref[1] + y1_ref[...]
        h = jnp.maximum(ssum * dinv_ref[...] + b1_ref[...], 0.0)
        y2_ref[...] = jnp.dot(h, w2_ref[...],
                              preferred_element_type=jnp.float32) * dinv_ref[...]

    return pl.pallas_call(
        body,
        grid=(n // bn,),
        in_specs=[
            pl.BlockSpec((2, bn, hid), lambda i: (0, i, 0)),
            pl.BlockSpec((bn, hid), lambda i: (i, 0)),
            pl.BlockSpec((bn, 1), lambda i: (i, 0)),
            pl.BlockSpec((1, hid), lambda i: (0, 0)),
            pl.BlockSpec((hid, c), lambda i: (0, 0)),
        ],
        out_specs=pl.BlockSpec((bn, c), lambda i: (i, 0)),
        out_shape=jax.ShapeDtypeStruct((n, c), jnp.float32),
    )(aggp, y1, dinv, b1, W2)


def _tc_out(aggp, y2, dinv, b2, bn):
    n, c = y2.shape

    def body(aggp_ref, y2_ref, dinv_ref, b2_ref, out_ref):
        o = (aggp_ref[0] + aggp_ref[1] + y2_ref[...]) * dinv_ref[...] \
            + b2_ref[...]
        m = jnp.max(o, axis=1, keepdims=True)
        e = jnp.exp(o - m)
        lse = jnp.log(jnp.sum(e, axis=1, keepdims=True))
        out_ref[...] = o - m - lse

    return pl.pallas_call(
        body,
        grid=(n // bn,),
        in_specs=[
            pl.BlockSpec((2, bn, c), lambda i: (0, i, 0)),
            pl.BlockSpec((bn, c), lambda i: (i, 0)),
            pl.BlockSpec((bn, 1), lambda i: (i, 0)),
            pl.BlockSpec((1, c), lambda i: (0, 0)),
        ],
        out_specs=pl.BlockSpec((bn, c), lambda i: (i, 0)),
        out_shape=jax.ShapeDtypeStruct((n, c), jnp.float32),
    )(aggp, y2, dinv, b2)


def kernel(x, edge_index, W1, b1, W2, b2):
    n, din = x.shape
    hid = W1.shape[1]
    nc = W2.shape[1]
    e = edge_index.shape[1]

    ei = edge_index.astype(jnp.int32)
    nch = e // CHUNK
    if nch * CHUNK != e:
        nch += 1
        ei = jnp.concatenate(
            [ei, jnp.broadcast_to(
                jnp.array([[0], [n]], jnp.int32),
                (2, nch * CHUNK - e))], axis=1)
    e3 = ei.reshape(2, nch, CHUNK)
    base = nch // NW
    extra = nch - base * NW

    np_rows = -(-(n + 1) // 256) * 256
    zeros1 = jnp.zeros((np_rows,), jnp.float32)
    zerosh = jnp.zeros((np_rows, hid), jnp.float32)
    zerosc = jnp.zeros((np_rows, nc), jnp.float32)

    bn = 1000 if n % 1000 == 0 else 8

    degp = _sc_degree(e3, zeros1, np_rows, base, extra)
    y1, dinv = _tc_prep(x, degp[:, :n].T, W1, bn)
    agg1 = _sc_agg(y1, e3, zerosh, np_rows, base, extra, hid)
    y2 = _tc_layer1(agg1, y1, dinv, b1.reshape(1, hid), W2, bn)
    agg2 = _sc_agg(y2, e3, zerosc, np_rows, base, extra, nc)
    return _tc_out(agg2, y2, dinv, b2.reshape(1, nc), bn)


# speedup vs baseline: 61.5675x; 1.0294x over previous
import functools

import jax
import jax.numpy as jnp
from jax import lax
from jax.experimental import pallas as pl
from jax.experimental.pallas import tpu as pltpu
from jax.experimental.pallas import tpu_sc as plsc

NC = 2
NS = 16
NW = NC * NS
CHUNK = 128


def _ring(base):
    for d in (8, 6, 4, 2):
        if base % d == 0 and base // d >= 2:
            return d
    return 1


def _sc_degree(e3, zeros1, np_rows, base, extra):
    stripe = np_rows // NS
    D = _ring(base)
    mesh = plsc.VectorSubcoreMesh(core_axis_name="c", subcore_axis_name="s")

    @functools.partial(
        pl.kernel,
        mesh=mesh,
        compiler_params=pltpu.CompilerParams(use_tc_tiling_on_sc=False),
        out_type=jax.ShapeDtypeStruct((NC, np_rows), jnp.float32),
        scratch_types=[
            pltpu.VMEM((base, CHUNK), jnp.int32),
            pltpu.VMEM((1, CHUNK), jnp.int32),
            pltpu.VMEM((CHUNK,), jnp.float32),
            pltpu.VMEM_SHARED((np_rows,), jnp.float32),
        ] + [pltpu.SemaphoreType.DMA] * max(D, 1),
    )
    def kfn(zeros_hbm, e3_hbm, out_hbm, dst_v, dstx_v, ones_v, acc_sh,
            *sems):
        c = lax.axis_index("c")
        s = lax.axis_index("s")
        w = c * NS + s
        pltpu.sync_copy(zeros_hbm.at[pl.ds(s * stripe, stripe)],
                        acc_sh.at[pl.ds(s * stripe, stripe)])
        for i in range(CHUNK // 16):
            ones_v[pl.ds(i * 16, 16)] = jnp.ones((16,), jnp.float32)
        pltpu.sync_copy(e3_hbm.at[1, pl.ds(w * base, base)], dst_v)
        plsc.subcore_barrier()

        def start(i, b):
            pltpu.async_copy(ones_v, acc_sh.at[dst_v.at[i]], sems[b],
                             add=True)

        def drain(b):
            pltpu.make_async_copy(ones_v, acc_sh.at[dst_v.at[0]],
                                  sems[b]).wait()

        for i in range(D):
            start(i, i)

        def group(jo, carry):
            for b in range(D):
                drain(b)
                start(jo * D + b, b)
            return carry

        lax.fori_loop(1, base // D, group, 0)
        for b in range(D):
            drain(b)
        @pl.when(w < extra)
        def _():
            pltpu.sync_copy(e3_hbm.at[1, pl.ds(base * NW + w, 1)], dstx_v)
            pltpu.sync_copy(ones_v, acc_sh.at[dstx_v.at[0]], add=True)

        plsc.subcore_barrier()
        pltpu.sync_copy(acc_sh.at[pl.ds(s * stripe, stripe)],
                        out_hbm.at[c, pl.ds(s * stripe, stripe)])

    return kfn(zeros1, e3)


def _sc_agg(y, e3, zerosd, np_rows, base, extra, d):
    stripe = np_rows // NS
    D = _ring(base)
    A = min(3, max(D - 1, 1))
    mesh = plsc.VectorSubcoreMesh(core_axis_name="c", subcore_axis_name="s")

    @functools.partial(
        pl.kernel,
        mesh=mesh,
        compiler_params=pltpu.CompilerParams(use_tc_tiling_on_sc=False),
        out_type=jax.ShapeDtypeStruct((NC, np_rows, d), jnp.float32),
        scratch_types=[
            pltpu.VMEM((base, CHUNK), jnp.int32),
            pltpu.VMEM((base, CHUNK), jnp.int32),
            pltpu.VMEM((1, CHUNK), jnp.int32),
            pltpu.VMEM((1, CHUNK), jnp.int32),
        ] + [pltpu.VMEM((CHUNK, d), jnp.float32)] * (max(D, 1) + 1)
          + [pltpu.VMEM_SHARED((np_rows, d), jnp.float32)] * 2
          + [pltpu.SemaphoreType.DMA] * (2 * max(D, 1) + 1),
    )
    def kfn(y_hbm, e3_hbm, zeros_hbm, out_hbm,
            src_v, dst_v, srcx_v, dstx_v, *bufs_and_sems):
        rows = bufs_and_sems[:D]
        rowsx = bufs_and_sems[D]
        acc_sh = bufs_and_sems[D + 1]
        y_sh = bufs_and_sems[D + 2]
        gsem = bufs_and_sems[D + 3:D + 3 + D]
        ssem = bufs_and_sems[D + 3 + D:D + 3 + 2 * D]
        xsem = bufs_and_sems[D + 3 + 2 * D]
        n = y_hbm.shape[0]
        ystripe = n // NS
        c = lax.axis_index("c")
        s = lax.axis_index("s")
        w = c * NS + s
        cp0 = pltpu.async_copy(zeros_hbm.at[pl.ds(s * stripe, stripe)],
                               acc_sh.at[pl.ds(s * stripe, stripe)], gsem[0])
        cp1 = pltpu.async_copy(y_hbm.at[pl.ds(s * ystripe, ystripe)],
                               y_sh.at[pl.ds(s * ystripe, ystripe)],
                               gsem[min(1, D - 1)])
        cp2 = pltpu.async_copy(e3_hbm.at[0, pl.ds(w * base, base)], src_v,
                               ssem[0])
        cp3 = pltpu.async_copy(e3_hbm.at[1, pl.ds(w * base, base)], dst_v,
                               ssem[min(1, D - 1)])
        cp0.wait()
        cp1.wait()
        cp2.wait()
        cp3.wait()
        plsc.subcore_barrier()

        def start_gather(i, b):
            pltpu.async_copy(y_sh.at[src_v.at[i]], rows[b], gsem[b])

        def wait_gather(i, b):
            pltpu.make_async_copy(y_sh.at[src_v.at[i]], rows[b],
                                  gsem[b]).wait()

        def start_scatter(i, b):
            pltpu.async_copy(rows[b], acc_sh.at[dst_v.at[i]], ssem[b],
                             add=True)

        def drain_scatter(b):
            pltpu.make_async_copy(rows[b], acc_sh.at[dst_v.at[0]],
                                  ssem[b]).wait()

        for i in range(D):
            start_gather(i, i)
            if i >= A:
                wait_gather(i - A, i - A)
                start_scatter(i - A, i - A)

        def group(jo, carry):
            for b in range(D):
                i = jo * D + b
                drain_scatter(b)
                start_gather(i, b)
                bA = (b - A) % D
                wait_gather(i - A, bA)
                start_scatter(i - A, bA)
            return carry

        lax.fori_loop(1, base // D, group, 0)
        for t in range(A):
            i = base - A + t
            b = i % D
            wait_gather(i, b)
            start_scatter(i, b)
        for b in range(D):
            drain_scatter(b)
        @pl.when(w < extra)
        def _():
            pltpu.sync_copy(e3_hbm.at[0, pl.ds(base * NW + w, 1)], srcx_v)
            pltpu.sync_copy(e3_hbm.at[1, pl.ds(base * NW + w, 1)], dstx_v)
            pltpu.async_copy(y_sh.at[srcx_v.at[0]], rowsx, xsem).wait()
            pltpu.sync_copy(rowsx, acc_sh.at[dstx_v.at[0]], add=True)

        plsc.subcore_barrier()
        pltpu.sync_copy(acc_sh.at[pl.ds(s * stripe, stripe)],
                        out_hbm.at[c, pl.ds(s * stripe, stripe)])

    return kfn(y, e3, zerosd)


def _tc_prep(x, degp, W1, bn):
    n, din = x.shape
    hid = W1.shape[1]

    def body(x_ref, degp_ref, w1_ref, y1_ref, dinv_ref):
        deg = degp_ref[:, 0:1] + degp_ref[:, 1:2] + 1.0
        dinv = lax.rsqrt(deg)
        xw = jnp.dot(x_ref[...], w1_ref[...],
                     preferred_element_type=jnp.float32)
        y1_ref[...] = xw * dinv
        dinv_ref[...] = dinv

    return pl.pallas_call(
        body,
        grid=(n // bn,),
        in_specs=[
            pl.BlockSpec((bn, din), lambda i: (i, 0)),
            pl.BlockSpec((bn, 2), lambda i: (i, 0)),
            pl.BlockSpec((din, hid), lambda i: (0, 0)),
        ],
        out_specs=[
            pl.BlockSpec((bn, hid), lambda i: (i, 0)),
            pl.BlockSpec((bn, 1), lambda i: (i, 0)),
        ],
        out_shape=[
            jax.ShapeDtypeStruct((n, hid), jnp.float32),
            jax.ShapeDtypeStruct((n, 1), jnp.float32),
        ],
    )(x, degp, W1)


def _tc_layer1(aggp, y1, dinv, b1, W2, bn):
    n, hid = y1.shape
    c = W2.shape[1]

    def body(aggp_ref, y1_ref, dinv_ref, b1_ref, w2_ref, y2_ref):
        ssum = aggp_ref[0] + aggp_ref[1] + y1_ref[...]
        h = jnp.maximum(ssum * dinv_ref[...] + b1_ref[...], 0.0)
        y2_ref[...] = jnp.dot(h, w2_ref[...],
                              preferred_element_type=jnp.float32) * dinv_ref[...]

    return pl.pallas_call(
        body,
        grid=(n // bn,),
        in_specs=[
            pl.BlockSpec((2, bn, hid), lambda i: (0, i, 0)),
            pl.BlockSpec((bn, hid), lambda i: (i, 0)),
            pl.BlockSpec((bn, 1), lambda i: (i, 0)),
            pl.BlockSpec((1, hid), lambda i: (0, 0)),
            pl.BlockSpec((hid, c), lambda i: (0, 0)),
        ],
        out_specs=pl.BlockSpec((bn, c), lambda i: (i, 0)),
        out_shape=jax.ShapeDtypeStruct((n, c), jnp.float32),
    )(aggp, y1, dinv, b1, W2)


def _tc_out(aggp, y2, dinv, b2, bn):
    n, c = y2.shape

    def body(aggp_ref, y2_ref, dinv_ref, b2_ref, out_ref):
        o = (aggp_ref[0] + aggp_ref[1] + y2_ref[...]) * dinv_ref[...] \
            + b2_ref[...]
        m = jnp.max(o, axis=1, keepdims=True)
        e = jnp.exp(o - m)
        lse = jnp.log(jnp.sum(e, axis=1, keepdims=True))
        out_ref[...] = o - m - lse

    return pl.pallas_call(
        body,
        grid=(n // bn,),
        in_specs=[
            pl.BlockSpec((2, bn, c), lambda i: (0, i, 0)),
            pl.BlockSpec((bn, c), lambda i: (i, 0)),
            pl.BlockSpec((bn, 1), lambda i: (i, 0)),
            pl.BlockSpec((1, c), lambda i: (0, 0)),
        ],
        out_specs=pl.BlockSpec((bn, c), lambda i: (i, 0)),
        out_shape=jax.ShapeDtypeStruct((n, c), jnp.float32),
    )(aggp, y2, dinv, b2)


def kernel(x, edge_index, W1, b1, W2, b2):
    n, din = x.shape
    hid = W1.shape[1]
    nc = W2.shape[1]
    e = edge_index.shape[1]

    ei = edge_index.astype(jnp.int32)
    nch = e // CHUNK
    if nch * CHUNK != e:
        nch += 1
        ei = jnp.concatenate(
            [ei, jnp.broadcast_to(
                jnp.array([[0], [n]], jnp.int32),
                (2, nch * CHUNK - e))], axis=1)
    e3 = ei.reshape(2, nch, CHUNK)
    base = nch // NW
    extra = nch - base * NW

    np_rows = -(-(n + 1) // 256) * 256
    zeros1 = jnp.zeros((np_rows,), jnp.float32)
    zerosh = jnp.zeros((np_rows, hid), jnp.float32)
    zerosc = jnp.zeros((np_rows, nc), jnp.float32)

    bn = 1000 if n % 1000 == 0 else 8

    degp = _sc_degree(e3, zeros1, np_rows, base, extra)
    y1, dinv = _tc_prep(x, degp[:, :n].T, W1, bn)
    agg1 = _sc_agg(y1, e3, zerosh, np_rows, base, extra, hid)
    y2 = _tc_layer1(agg1, y1, dinv, b1.reshape(1, hid), W2, bn)
    agg2 = _sc_agg(y2, e3, zerosc, np_rows, base, extra, nc)
    return _tc_out(agg2, y2, dinv, b2.reshape(1, nc), bn)
